# baseline (speedup 1.0000x reference)
import functools

import jax
import jax.numpy as jnp
from jax import lax
from jax.experimental import pallas as pl
from jax.experimental.pallas import tpu as pltpu
from jax.experimental.pallas import tpu_sc as plsc

_NW = 32
_SLAB = 3200
_NPAD = _NW * _SLAB
_GROUPS = _SLAB // 16
_BLK = 4096


def _make_route_body(n):
    tail_rows = n - (_NW - 1) * _SLAB
    tail_rows16 = ((tail_rows + 15) // 16) * 16

    def body(f0_hbm, f1_hbm, bt_hbm, ws_hbm, bs_hbm, wg_hbm, bg_hbm, wl_hbm,
             bl_hbm, out_hbm, wcat_hbm, f0_v, f1_v, bt_v, buf_v, w_v):
        wid = lax.axis_index("s") * 2 + lax.axis_index("c")
        base = wid * _SLAB
        last = wid == _NW - 1

        @pl.when(jnp.logical_not(last))
        def _():
            pltpu.sync_copy(f0_hbm.at[pl.ds(base, _SLAB)], f0_v)
            pltpu.sync_copy(f1_hbm.at[pl.ds(base, _SLAB)], f1_v)
            pltpu.sync_copy(bt_hbm.at[pl.ds(base, _SLAB)], bt_v)

        @pl.when(last)
        def _():
            pltpu.sync_copy(f0_hbm.at[pl.ds(base, tail_rows)],
                            f0_v.at[pl.ds(0, tail_rows)])
            pltpu.sync_copy(f1_hbm.at[pl.ds(base, tail_rows)],
                            f1_v.at[pl.ds(0, tail_rows)])
            pltpu.sync_copy(bt_hbm.at[pl.ds(base, tail_rows)],
                            bt_v.at[pl.ds(0, tail_rows)])

        @pl.when(wid == 0)
        def _():
            pltpu.sync_copy(ws_hbm, w_v.at[pl.ds(0, 2), :])
            pltpu.sync_copy(wg_hbm, w_v.at[pl.ds(2, 2), :])
            pltpu.sync_copy(wl_hbm, w_v.at[pl.ds(4, 2), :])
            pltpu.sync_copy(bs_hbm, w_v.at[pl.ds(6, 1), :])
            pltpu.sync_copy(bg_hbm, w_v.at[pl.ds(7, 1), :])
            pltpu.sync_copy(bl_hbm, w_v.at[pl.ds(8, 1), :])
            pltpu.sync_copy(w_v, wcat_hbm)

        iot = lax.broadcasted_iota(jnp.int32, (16,), 0)
        zero_f = jnp.zeros((16,), jnp.float32)
        one_f = jnp.ones((16,), jnp.float32)
        nvalid = jnp.where(last, tail_rows16, _SLAB)

        def group(k, carry):
            r = k * 16
            tv = bt_v[pl.ds(r, 16)]
            f0 = f0_v[pl.ds(r, 16)]
            f1 = f1_v[pl.ds(r, 16)]
            ok = (base + r + iot) < n
            m1 = (tv == 1) & ok
            m2 = (tv == 2) & ok
            m3 = (tv == 3) & ok
            buf_v[0, pl.ds(r, 16)] = jnp.where(m1, f0, zero_f)
            buf_v[1, pl.ds(r, 16)] = jnp.where(m1, f1, zero_f)
            buf_v[2, pl.ds(r, 16)] = jnp.where(m2, f0, zero_f)
            buf_v[3, pl.ds(r, 16)] = jnp.where(m2, f1, zero_f)
            buf_v[4, pl.ds(r, 16)] = jnp.where(m3, f0, zero_f)
            buf_v[5, pl.ds(r, 16)] = jnp.where(m3, f1, zero_f)
            buf_v[6, pl.ds(r, 16)] = jnp.where(m1, one_f, zero_f)
            buf_v[7, pl.ds(r, 16)] = jnp.where(m2, one_f, zero_f)
            buf_v[8, pl.ds(r, 16)] = jnp.where(m3, one_f, zero_f)
            return carry

        lax.fori_loop(0, nvalid // 16, group, 0)

        def zgroup(k, carry):
            r = k * 16
            for c in range(9):
                buf_v[c, pl.ds(r, 16)] = zero_f
            return carry

        lax.fori_loop(nvalid // 16, _GROUPS, zgroup, 0)
        pltpu.sync_copy(buf_v, out_hbm.at[:, pl.ds(base, _SLAB)])

    return body


def _route(f0a, f1a, btype, ws, bs_r, wg, bg_r, wl, bl_r):
    n = btype.shape[0]
    d = ws.shape[1]
    mesh = plsc.VectorSubcoreMesh(core_axis_name="c", subcore_axis_name="s")
    fn = functools.partial(
        pl.kernel,
        mesh=mesh,
        out_type=(
            jax.ShapeDtypeStruct((9, _NPAD), jnp.float32),
            jax.ShapeDtypeStruct((9, d), jnp.float32),
        ),
        scratch_types=[
            pltpu.VMEM((_SLAB,), jnp.float32),
            pltpu.VMEM((_SLAB,), jnp.float32),
            pltpu.VMEM((_SLAB,), jnp.int32),
            pltpu.VMEM((9, _SLAB), jnp.float32),
            pltpu.VMEM((9, d), jnp.float32),
        ],
    )(_make_route_body(n))
    return fn(f0a, f1a, btype, ws, bs_r, wg, bg_r, wl, bl_r)


def _dense_body(fc_ref, w_ref, out_ref):
    pre = lax.dot_general(
        fc_ref[...], w_ref[...],
        (((0,), (0,)), ((), ())),
        preferred_element_type=jnp.float32,
    )
    out_ref[...] = jnp.tanh(pre)


@jax.jit
def kernel(feat, btype, Ws, bs, Wg, bg, Wl, bl):
    n, _ = feat.shape
    d = Ws.shape[1]
    fct, wcat = _route(
        feat[:, 0], feat[:, 1], btype,
        Ws, bs.reshape(1, d), Wg, bg.reshape(1, d), Wl, bl.reshape(1, d),
    )
    return pl.pallas_call(
        _dense_body,
        grid=((n + _BLK - 1) // _BLK,),
        in_specs=[
            pl.BlockSpec((9, _BLK), lambda i: (0, i)),
            pl.BlockSpec((9, d), lambda i: (0, 0)),
        ],
        out_specs=pl.BlockSpec((_BLK, d), lambda i: (i, 0)),
        out_shape=jax.ShapeDtypeStruct((n, d), jnp.float32),
    )(fct, wcat)


# speedup vs baseline: 1.0443x; 1.0443x over previous
import functools

import jax
import jax.numpy as jnp
from jax import lax
from jax.experimental import pallas as pl
from jax.experimental.pallas import tpu as pltpu
from jax.experimental.pallas import tpu_sc as plsc

_NW = 32
_SLAB = 3328
_NPAD = _NW * _SLAB
_GROUPS = _SLAB // 16
_HALF = _SLAB // 2
_BLK = 8192


def _route_body(f0_hbm, f1_hbm, bt_hbm, out_hbm, f0_v, f1_v, bt_v, buf_v,
                sem1, sem2):
    wid = lax.axis_index("s") * 2 + lax.axis_index("c")
    base = wid * _SLAB
    pltpu.sync_copy(f0_hbm.at[pl.ds(base, _SLAB)], f0_v)
    pltpu.sync_copy(f1_hbm.at[pl.ds(base, _SLAB)], f1_v)
    pltpu.sync_copy(bt_hbm.at[pl.ds(base, _SLAB)], bt_v)

    zero_f = jnp.zeros((16,), jnp.float32)
    one_f = jnp.ones((16,), jnp.float32)

    def group(k, carry):
        r = k * 16
        tv = bt_v[pl.ds(r, 16)]
        f0 = f0_v[pl.ds(r, 16)]
        f1 = f1_v[pl.ds(r, 16)]
        m1 = tv == 1
        m2 = tv == 2
        m3 = tv == 3
        buf_v[0, pl.ds(r, 16)] = jnp.where(m1, f0, zero_f)
        buf_v[1, pl.ds(r, 16)] = jnp.where(m1, f1, zero_f)
        buf_v[2, pl.ds(r, 16)] = jnp.where(m2, f0, zero_f)
        buf_v[3, pl.ds(r, 16)] = jnp.where(m2, f1, zero_f)
        buf_v[4, pl.ds(r, 16)] = jnp.where(m3, f0, zero_f)
        buf_v[5, pl.ds(r, 16)] = jnp.where(m3, f1, zero_f)
        buf_v[6, pl.ds(r, 16)] = jnp.where(m1, one_f, zero_f)
        buf_v[7, pl.ds(r, 16)] = jnp.where(m2, one_f, zero_f)
        buf_v[8, pl.ds(r, 16)] = jnp.where(m3, one_f, zero_f)
        return carry

    lax.fori_loop(0, _GROUPS // 2, group, 0)
    cp1 = pltpu.async_copy(
        buf_v.at[:, pl.ds(0, _HALF)],
        out_hbm.at[:, pl.ds(base, _HALF)], sem1)
    lax.fori_loop(_GROUPS // 2, _GROUPS, group, 0)
    cp2 = pltpu.async_copy(
        buf_v.at[:, pl.ds(_HALF, _HALF)],
        out_hbm.at[:, pl.ds(base + _HALF, _HALF)], sem2)
    cp1.wait()
    cp2.wait()


def _route(f0a, f1a, bt_pad):
    mesh = plsc.VectorSubcoreMesh(core_axis_name="c", subcore_axis_name="s")
    fn = functools.partial(
        pl.kernel,
        mesh=mesh,
        out_type=jax.ShapeDtypeStruct((9, _NPAD), jnp.float32),
        scratch_types=[
            pltpu.VMEM((_SLAB,), jnp.float32),
            pltpu.VMEM((_SLAB,), jnp.float32),
            pltpu.VMEM((_SLAB,), jnp.int32),
            pltpu.VMEM((9, _SLAB), jnp.float32),
            pltpu.SemaphoreType.DMA,
            pltpu.SemaphoreType.DMA,
        ],
    )(_route_body)
    return fn(f0a, f1a, bt_pad)


def _dense_body(fc_ref, w_ref, out_ref):
    pre = lax.dot_general(
        fc_ref[...], w_ref[...],
        (((0,), (0,)), ((), ())),
        preferred_element_type=jnp.float32,
    )
    out_ref[...] = jnp.tanh(pre)


@jax.jit
def kernel(feat, btype, Ws, bs, Wg, bg, Wl, bl):
    n, _ = feat.shape
    d = Ws.shape[1]
    wcat = jnp.zeros((9, d), jnp.float32)
    wcat = wcat.at[0:2].set(Ws).at[2:4].set(Wg).at[4:6].set(Wl)
    wcat = wcat.at[6].set(bs).at[7].set(bg).at[8].set(bl)

    feat_pad = jnp.pad(feat, ((0, _NPAD - n), (0, 0)))
    bt_pad = jnp.pad(btype, (0, _NPAD - n))

    fct = _route(feat_pad[:, 0], feat_pad[:, 1], bt_pad)

    return pl.pallas_call(
        _dense_body,
        grid=((n + _BLK - 1) // _BLK,),
        in_specs=[
            pl.BlockSpec((9, _BLK), lambda i: (0, i)),
            pl.BlockSpec((9, d), lambda i: (0, 0)),
        ],
        out_specs=pl.BlockSpec((_BLK, d), lambda i: (i, 0)),
        out_shape=jax.ShapeDtypeStruct((n, d), jnp.float32),
    )(fct, wcat)


# speedup vs baseline: 1.0549x; 1.0101x over previous
import functools

import jax
import jax.numpy as jnp
from jax import lax
from jax.experimental import pallas as pl
from jax.experimental.pallas import tpu as pltpu
from jax.experimental.pallas import tpu_sc as plsc

_NW = 32
_SLAB = 3328
_NPAD = _NW * _SLAB
_GROUPS = _SLAB // 16
_HALF = _SLAB // 2
_BLK = 4096


def _route_body(f0_hbm, f1_hbm, bt_hbm, out_hbm, f0_v, f1_v, bt_v, buf_v,
                sem1, sem2):
    wid = lax.axis_index("s") * 2 + lax.axis_index("c")
    base = wid * _SLAB
    pltpu.sync_copy(f0_hbm.at[pl.ds(base, _SLAB)], f0_v)
    pltpu.sync_copy(f1_hbm.at[pl.ds(base, _SLAB)], f1_v)
    pltpu.sync_copy(bt_hbm.at[pl.ds(base, _SLAB)], bt_v)

    zero_f = jnp.zeros((16,), jnp.float32)
    one_f = jnp.ones((16,), jnp.float32)

    def group(k, carry):
        r = k * 16
        tv = bt_v[pl.ds(r, 16)]
        f0 = f0_v[pl.ds(r, 16)]
        f1 = f1_v[pl.ds(r, 16)]
        m1 = tv == 1
        m2 = tv == 2
        m3 = tv == 3
        buf_v[0, pl.ds(r, 16)] = jnp.where(m1, f0, zero_f)
        buf_v[1, pl.ds(r, 16)] = jnp.where(m1, f1, zero_f)
        buf_v[2, pl.ds(r, 16)] = jnp.where(m2, f0, zero_f)
        buf_v[3, pl.ds(r, 16)] = jnp.where(m2, f1, zero_f)
        buf_v[4, pl.ds(r, 16)] = jnp.where(m3, f0, zero_f)
        buf_v[5, pl.ds(r, 16)] = jnp.where(m3, f1, zero_f)
        buf_v[6, pl.ds(r, 16)] = jnp.where(m1, one_f, zero_f)
        buf_v[7, pl.ds(r, 16)] = jnp.where(m2, one_f, zero_f)
        buf_v[8, pl.ds(r, 16)] = jnp.where(m3, one_f, zero_f)
        return carry

    lax.fori_loop(0, _GROUPS // 2, group, 0)
    cp1 = pltpu.async_copy(
        buf_v.at[:, pl.ds(0, _HALF)],
        out_hbm.at[:, pl.ds(base, _HALF)], sem1)
    lax.fori_loop(_GROUPS // 2, _GROUPS, group, 0)
    cp2 = pltpu.async_copy(
        buf_v.at[:, pl.ds(_HALF, _HALF)],
        out_hbm.at[:, pl.ds(base + _HALF, _HALF)], sem2)
    cp1.wait()
    cp2.wait()


def _route(f0a, f1a, bt_pad):
    mesh = plsc.VectorSubcoreMesh(core_axis_name="c", subcore_axis_name="s")
    fn = functools.partial(
        pl.kernel,
        mesh=mesh,
        out_type=jax.ShapeDtypeStruct((9, _NPAD), jnp.float32),
        scratch_types=[
            pltpu.VMEM((_SLAB,), jnp.float32),
            pltpu.VMEM((_SLAB,), jnp.float32),
            pltpu.VMEM((_SLAB,), jnp.int32),
            pltpu.VMEM((9, _SLAB), jnp.float32),
            pltpu.SemaphoreType.DMA,
            pltpu.SemaphoreType.DMA,
        ],
    )(_route_body)
    return fn(f0a, f1a, bt_pad)


def _dense_body(fc_ref, w_ref, out_ref):
    pre = lax.dot_general(
        fc_ref[...], w_ref[...],
        (((0,), (0,)), ((), ())),
        preferred_element_type=jnp.float32,
    )
    out_ref[...] = jnp.tanh(pre)


@jax.jit
def kernel(feat, btype, Ws, bs, Wg, bg, Wl, bl):
    n, _ = feat.shape
    d = Ws.shape[1]
    wcat = jnp.zeros((9, d), jnp.float32)
    wcat = wcat.at[0:2].set(Ws).at[2:4].set(Wg).at[4:6].set(Wl)
    wcat = wcat.at[6].set(bs).at[7].set(bg).at[8].set(bl)

    feat_pad = jnp.pad(feat, ((0, _NPAD - n), (0, 0)))
    bt_pad = jnp.pad(btype, (0, _NPAD - n))

    fct = _route(feat_pad[:, 0], feat_pad[:, 1], bt_pad)

    return pl.pallas_call(
        _dense_body,
        grid=((n + _BLK - 1) // _BLK,),
        in_specs=[
            pl.BlockSpec((9, _BLK), lambda i: (0, i)),
            pl.BlockSpec((9, d), lambda i: (0, 0)),
        ],
        out_specs=pl.BlockSpec((_BLK, d), lambda i: (i, 0)),
        out_shape=jax.ShapeDtypeStruct((n, d), jnp.float32),
    )(fct, wcat)
